# Initial kernel scaffold; baseline (speedup 1.0000x reference)
import jax
import jax.numpy as jnp
from jax.experimental import pallas as pl


def kernel(x, c_sl1, c_sl2, c_sl3, c_r1, c_r2, c_r3, w_sl1, b_sl1, g_sl1, be_sl1, w_sl2, b_sl2, g_sl2, be_sl2, w_sl3, b_sl3, g_sl3, be_sl3, w_r1, b_r1, g_r1, be_r1, w_r2, b_r2, g_r2, be_r2, w_r3, b_r3, g_r3, be_r3):
    raise NotImplementedError("write your pallas kernel here")


# speedup vs baseline: 1.8268x; 1.8268x over previous
import jax
import jax.numpy as jnp
from jax.experimental import pallas as pl

_SQRT1_2 = 0.7071067811865476
_EPS = 1e-5


def _dense_from_edges(c0, c1, w, din, dout):
    e = w.shape[-1]
    a = jnp.where(jax.lax.broadcasted_iota(jnp.int32, (din, e), 0) == c1, w, 0.0)
    bt = (jax.lax.broadcasted_iota(jnp.int32, (dout, e), 0) == c0).astype(jnp.float32)
    return jax.lax.dot_general(
        a, bt, (((1,), (1,)), ((), ())),
        preferred_element_type=jnp.float32,
        precision=jax.lax.Precision.HIGHEST,
    )


def _densify_kernel(c0_1, c1_1, w_1, c0_2, c1_2, w_2, c0_3, c1_3, w_3,
                    c0_r1, c1_r1, w_r1, c0_r2, c1_r2, w_r2, c0_r3, c1_r3, w_r3,
                    m1, m2, m3, mr1, mr2, mr3):
    m1[...] = _dense_from_edges(c0_1[...], c1_1[...], w_1[...], 32, 48)
    m2[...] = _dense_from_edges(c0_2[...], c1_2[...], w_2[...], 48, 32)
    m3[...] = _dense_from_edges(c0_3[...], c1_3[...], w_3[...], 32, 8)
    mr1[...] = _dense_from_edges(c0_r1[...], c1_r1[...], w_r1[...], 48, 32)
    mr2[...] = _dense_from_edges(c0_r2[...], c1_r2[...], w_r2[...], 32, 32)
    mr3[...] = _dense_from_edges(c0_r3[...], c1_r3[...], w_r3[...], 8, 32)


def _ln_gelu(y, g, be):
    mu = jnp.mean(y, axis=-1, keepdims=True)
    d = y - mu
    var = jnp.mean(d * d, axis=-1, keepdims=True)
    y = d * jax.lax.rsqrt(var + _EPS) * g + be
    return 0.5 * y * (1.0 + jax.lax.erf(y * _SQRT1_2))


def _main_kernel(x_ref, m1, m2, m3, mr1, mr2, mr3,
                 b1, g1, be1, b2, g2, be2, b3, g3, be3,
                 br1, gr1, ber1, br2, gr2, ber2, br3, gr3, ber3, out_ref):
    def dot(a, m):
        return jax.lax.dot_general(
            a, m[...], (((1,), (0,)), ((), ())),
            preferred_element_type=jnp.float32,
            precision=jax.lax.Precision.HIGHEST,
        )

    x = x_ref[...]
    s1 = _ln_gelu(dot(x, m1) + b1[...], g1[...], be1[...])
    s2 = _ln_gelu(dot(s1, m2) + b2[...], g2[...], be2[...])
    s3 = _ln_gelu(dot(s2, m3) + b3[...], g3[...], be3[...])
    o = (_ln_gelu(dot(s1, mr1) + br1[...], gr1[...], ber1[...])
         + _ln_gelu(dot(s2, mr2) + br2[...], gr2[...], ber2[...])
         + _ln_gelu(dot(s3, mr3) + br3[...], gr3[...], ber3[...]))
    out_ref[...] = o


_BLOCK_ROWS = 2048


def kernel(x, c_sl1, c_sl2, c_sl3, c_r1, c_r2, c_r3,
           w_sl1, b_sl1, g_sl1, be_sl1, w_sl2, b_sl2, g_sl2, be_sl2,
           w_sl3, b_sl3, g_sl3, be_sl3, w_r1, b_r1, g_r1, be_r1,
           w_r2, b_r2, g_r2, be_r2, w_r3, b_r3, g_r3, be_r3):
    orig_shape = x.shape
    n = x.shape[0] * x.shape[1]
    x2 = x.reshape(n, 32)

    def rows(c):
        c = c.astype(jnp.int32)
        return c[0:1, :], c[1:2, :]

    dims = {"sl1": (32, 48), "sl2": (48, 32), "sl3": (32, 8),
            "r1": (48, 32), "r2": (32, 32), "r3": (8, 32)}
    conns = {"sl1": c_sl1, "sl2": c_sl2, "sl3": c_sl3,
             "r1": c_r1, "r2": c_r2, "r3": c_r3}
    ws = {"sl1": w_sl1, "sl2": w_sl2, "sl3": w_sl3,
          "r1": w_r1, "r2": w_r2, "r3": w_r3}

    densify_in = []
    for k in ("sl1", "sl2", "sl3", "r1", "r2", "r3"):
        c0, c1 = rows(conns[k])
        densify_in += [c0, c1, ws[k].reshape(1, -1)]

    mats = pl.pallas_call(
        _densify_kernel,
        out_shape=[jax.ShapeDtypeStruct(dims[k], jnp.float32)
                   for k in ("sl1", "sl2", "sl3", "r1", "r2", "r3")],
    )(*densify_in)

    scalars = []
    for b, g, be in ((b_sl1, g_sl1, be_sl1), (b_sl2, g_sl2, be_sl2),
                     (b_sl3, g_sl3, be_sl3), (b_r1, g_r1, be_r1),
                     (b_r2, g_r2, be_r2), (b_r3, g_r3, be_r3)):
        scalars += [b.reshape(1, -1), g.reshape(1, -1), be.reshape(1, -1)]

    grid = n // _BLOCK_ROWS
    full = lambda shape: pl.BlockSpec(shape, lambda i: (0, 0))
    out = pl.pallas_call(
        _main_kernel,
        grid=(grid,),
        in_specs=[pl.BlockSpec((_BLOCK_ROWS, 32), lambda i: (i, 0))]
        + [full(dims[k]) for k in ("sl1", "sl2", "sl3", "r1", "r2", "r3")]
        + [full(s.shape) for s in scalars],
        out_specs=pl.BlockSpec((_BLOCK_ROWS, 32), lambda i: (i, 0)),
        out_shape=jax.ShapeDtypeStruct((n, 32), jnp.float32),
    )(x2, *mats, *scalars)
    return out.reshape(orig_shape)
